# Initial kernel scaffold; baseline (speedup 1.0000x reference)
import jax
import jax.numpy as jnp
from jax.experimental import pallas as pl


def kernel(h_t, W1, b1, W2, b2, codebook, threshold):
    raise NotImplementedError("write your pallas kernel here")


# speedup vs baseline: 8.3478x; 8.3478x over previous
import jax
import jax.numpy as jnp
from jax.experimental import pallas as pl
from jax.experimental.pallas import tpu as pltpu

_T = 3
_LANES = 128
_BB = 512


def _router_block(thr_ref, x_ref, w1_ref, b1_ref, w2_ref, b2_ref, cb_ref,
                  e_ref, g_ref):
    x = x_ref[...]
    h = jax.lax.dot_general(x, w1_ref[...], (((1,), (0,)), ((), ())),
                            preferred_element_type=jnp.float32)
    h = jnp.maximum(h + b1_ref[...], 0.0)
    logits = jax.lax.dot_general(h, w2_ref[...], (((1,), (0,)), ((), ())),
                                 preferred_element_type=jnp.float32)
    logits = logits + b2_ref[...]

    m = jnp.max(logits, axis=-1, keepdims=True)
    ex = jnp.exp(logits - m)
    s = jnp.sum(ex, axis=-1, keepdims=True)
    p = ex / s

    j = jax.lax.broadcasted_iota(jnp.int32, p.shape, 1)
    valid = j < _T
    maxp = jnp.max(p, axis=-1, keepdims=True)
    minp = jnp.min(jnp.where(valid, p, jnp.inf), axis=-1, keepdims=True)
    a = jnp.min(jnp.where(p == maxp, j, _LANES), axis=-1, keepdims=True)
    l = jnp.max(jnp.where(valid & (p == minp), j, -1), axis=-1, keepdims=True)

    use_topk = maxp < thr_ref[0]
    w_topk = jnp.where(j == l, 0.0, p)
    w_arg = jnp.where(j == a, 1.0, 0.0)
    w = jnp.where(use_topk, w_topk, w_arg)

    e_ref[...] = jax.lax.dot_general(w, cb_ref[...], (((1,), (0,)), ((), ())),
                                     preferred_element_type=jnp.float32)
    g_ref[...] = p[:, :_T]


@jax.jit
def kernel(h_t, W1, b1, W2, b2, codebook, threshold=0.7):
    B, d_model = h_t.shape
    hidden = W1.shape[1]
    grid = B // _BB

    w2p = jnp.zeros((hidden, _LANES), jnp.float32).at[:, :_T].set(W2)
    b2p = jnp.full((1, _LANES), -1e30, jnp.float32).at[0, :_T].set(b2)
    cbp = jnp.zeros((_LANES, d_model), jnp.float32).at[:_T, :].set(codebook)
    thr = jnp.reshape(jnp.asarray(threshold, jnp.float32), (1,))

    e_task, g_task = pl.pallas_call(
        _router_block,
        grid=(grid,),
        in_specs=[
            pl.BlockSpec(memory_space=pltpu.SMEM),
            pl.BlockSpec((_BB, d_model), lambda i: (i, 0)),
            pl.BlockSpec((d_model, hidden), lambda i: (0, 0)),
            pl.BlockSpec((1, hidden), lambda i: (0, 0)),
            pl.BlockSpec((hidden, _LANES), lambda i: (0, 0)),
            pl.BlockSpec((1, _LANES), lambda i: (0, 0)),
            pl.BlockSpec((_LANES, d_model), lambda i: (0, 0)),
        ],
        out_specs=[
            pl.BlockSpec((_BB, d_model), lambda i: (i, 0)),
            pl.BlockSpec((_BB, _T), lambda i: (i, 0)),
        ],
        out_shape=[
            jax.ShapeDtypeStruct((B, d_model), jnp.float32),
            jax.ShapeDtypeStruct((B, _T), jnp.float32),
        ],
        compiler_params=pltpu.CompilerParams(
            dimension_semantics=("arbitrary",),
        ),
    )(thr, h_t, W1, jnp.reshape(b1, (1, hidden)), w2p, b2p, cbp)
    return (e_task, g_task)


# speedup vs baseline: 8.6998x; 1.0422x over previous
import jax
import jax.numpy as jnp
from jax.experimental import pallas as pl
from jax.experimental.pallas import tpu as pltpu

_T = 3
_LANES = 128
_BB = 1024


def _router_block(thr_ref, x_ref, w1_ref, b1_ref, w2_ref, b2_ref, cb_ref,
                  e_ref, g_ref):
    x = x_ref[...]
    h = jax.lax.dot_general(x, w1_ref[...], (((1,), (0,)), ((), ())),
                            preferred_element_type=jnp.float32)
    h = jnp.maximum(h + b1_ref[...], 0.0)
    logits = jax.lax.dot_general(h, w2_ref[...], (((1,), (0,)), ((), ())),
                                 preferred_element_type=jnp.float32)
    logits = logits + b2_ref[...]

    m = jnp.max(logits, axis=-1, keepdims=True)
    ex = jnp.exp(logits - m)
    s = jnp.sum(ex, axis=-1, keepdims=True)
    p = ex / s

    j = jax.lax.broadcasted_iota(jnp.int32, p.shape, 1)
    valid = j < _T
    maxp = jnp.max(p, axis=-1, keepdims=True)
    minp = jnp.min(jnp.where(valid, p, jnp.inf), axis=-1, keepdims=True)
    a = jnp.min(jnp.where(p == maxp, j, _LANES), axis=-1, keepdims=True)
    l = jnp.max(jnp.where(valid & (p == minp), j, -1), axis=-1, keepdims=True)

    use_topk = maxp < thr_ref[0]
    w_topk = jnp.where(j == l, 0.0, p)
    w_arg = jnp.where(j == a, 1.0, 0.0)
    w = jnp.where(use_topk, w_topk, w_arg)

    e_ref[...] = jax.lax.dot_general(w, cb_ref[...], (((1,), (0,)), ((), ())),
                                     preferred_element_type=jnp.float32)
    g_ref[...] = p[:, :_T]


@jax.jit
def kernel(h_t, W1, b1, W2, b2, codebook, threshold=0.7):
    B, d_model = h_t.shape
    hidden = W1.shape[1]
    grid = B // _BB

    w2p = jnp.zeros((hidden, _LANES), jnp.float32).at[:, :_T].set(W2)
    b2p = jnp.full((1, _LANES), -1e30, jnp.float32).at[0, :_T].set(b2)
    cbp = jnp.zeros((_LANES, d_model), jnp.float32).at[:_T, :].set(codebook)
    thr = jnp.reshape(jnp.asarray(threshold, jnp.float32), (1,))

    e_task, g_task = pl.pallas_call(
        _router_block,
        grid=(grid,),
        in_specs=[
            pl.BlockSpec(memory_space=pltpu.SMEM),
            pl.BlockSpec((_BB, d_model), lambda i: (i, 0)),
            pl.BlockSpec((d_model, hidden), lambda i: (0, 0)),
            pl.BlockSpec((1, hidden), lambda i: (0, 0)),
            pl.BlockSpec((hidden, _LANES), lambda i: (0, 0)),
            pl.BlockSpec((1, _LANES), lambda i: (0, 0)),
            pl.BlockSpec((_LANES, d_model), lambda i: (0, 0)),
        ],
        out_specs=[
            pl.BlockSpec((_BB, d_model), lambda i: (i, 0)),
            pl.BlockSpec((_BB, _T), lambda i: (i, 0)),
        ],
        out_shape=[
            jax.ShapeDtypeStruct((B, d_model), jnp.float32),
            jax.ShapeDtypeStruct((B, _T), jnp.float32),
        ],
        compiler_params=pltpu.CompilerParams(
            dimension_semantics=("arbitrary",),
        ),
    )(thr, h_t, W1, jnp.reshape(b1, (1, hidden)), w2p, b2p, cbp)
    return (e_task, g_task)


# speedup vs baseline: 8.7122x; 1.0014x over previous
import jax
import jax.numpy as jnp
from jax.experimental import pallas as pl
from jax.experimental.pallas import tpu as pltpu

_T = 3
_LANES = 128
_BB = 1024


def _router_block(thr_ref, x_ref, w1_ref, b1_ref, w2_ref, b2_ref, cb_ref,
                  e_ref, g_ref):
    x = x_ref[...]
    h = jax.lax.dot_general(x, w1_ref[...], (((1,), (0,)), ((), ())),
                            preferred_element_type=jnp.float32)
    h = jnp.maximum(h + b1_ref[...], 0.0)
    logits = jax.lax.dot_general(h, w2_ref[...], (((1,), (0,)), ((), ())),
                                 preferred_element_type=jnp.float32)
    logits = logits + b2_ref[...]

    m = jnp.max(logits, axis=-1, keepdims=True)
    ex = jnp.exp(logits - m)
    s = jnp.sum(ex, axis=-1, keepdims=True)
    p = ex / s

    j = jax.lax.broadcasted_iota(jnp.int32, p.shape, 1)
    valid = j < _T
    maxp = jnp.max(p, axis=-1, keepdims=True)
    minp = jnp.min(jnp.where(valid, p, jnp.inf), axis=-1, keepdims=True)
    a = jnp.min(jnp.where(p == maxp, j, _LANES), axis=-1, keepdims=True)
    l = jnp.max(jnp.where(valid & (p == minp), j, -1), axis=-1, keepdims=True)

    use_topk = maxp < thr_ref[0]
    w_topk = jnp.where(j == l, 0.0, p)
    w_arg = jnp.where(j == a, 1.0, 0.0)
    w = jnp.where(use_topk, w_topk, w_arg).astype(jnp.bfloat16)

    e_ref[...] = jax.lax.dot_general(w, cb_ref[...], (((1,), (0,)), ((), ())),
                                     preferred_element_type=jnp.float32)
    g_ref[...] = p[:, :_T]


@jax.jit
def kernel(h_t, W1, b1, W2, b2, codebook, threshold=0.7):
    B, d_model = h_t.shape
    hidden = W1.shape[1]
    grid = B // _BB

    w2p = jnp.zeros((hidden, _LANES), jnp.float32).at[:, :_T].set(W2)
    b2p = jnp.full((1, _LANES), -1e30, jnp.float32).at[0, :_T].set(b2)
    cbp = (jnp.zeros((_LANES, d_model), jnp.float32).at[:_T, :].set(codebook)
           .astype(jnp.bfloat16))
    thr = jnp.reshape(jnp.asarray(threshold, jnp.float32), (1,))

    e_task, g_task = pl.pallas_call(
        _router_block,
        grid=(grid,),
        in_specs=[
            pl.BlockSpec(memory_space=pltpu.SMEM),
            pl.BlockSpec((_BB, d_model), lambda i: (i, 0)),
            pl.BlockSpec((d_model, hidden), lambda i: (0, 0)),
            pl.BlockSpec((1, hidden), lambda i: (0, 0)),
            pl.BlockSpec((hidden, _LANES), lambda i: (0, 0)),
            pl.BlockSpec((1, _LANES), lambda i: (0, 0)),
            pl.BlockSpec((_LANES, d_model), lambda i: (0, 0)),
        ],
        out_specs=[
            pl.BlockSpec((_BB, d_model), lambda i: (i, 0)),
            pl.BlockSpec((_BB, _T), lambda i: (i, 0)),
        ],
        out_shape=[
            jax.ShapeDtypeStruct((B, d_model), jnp.float32),
            jax.ShapeDtypeStruct((B, _T), jnp.float32),
        ],
        compiler_params=pltpu.CompilerParams(
            dimension_semantics=("arbitrary",),
        ),
    )(thr, h_t, W1, jnp.reshape(b1, (1, hidden)), w2p, b2p, cbp)
    return (e_task, g_task)


# speedup vs baseline: 8.7126x; 1.0000x over previous
import jax
import jax.numpy as jnp
from jax.experimental import pallas as pl
from jax.experimental.pallas import tpu as pltpu

_T = 3
_LANES = 128
_BB = 1024


def _router_block(thr_ref, x_ref, w1_ref, b1_ref, w2_ref, b2_ref, cb_ref,
                  e_ref, g_ref):
    x = x_ref[...]
    h = jax.lax.dot_general(x, w1_ref[...], (((1,), (0,)), ((), ())),
                            preferred_element_type=jnp.float32)
    h = jnp.maximum(h + b1_ref[...], 0.0)
    logits = jax.lax.dot_general(h, w2_ref[...], (((1,), (0,)), ((), ())),
                                 preferred_element_type=jnp.float32)
    logits = logits + b2_ref[...]

    m = jnp.max(logits, axis=-1, keepdims=True)
    ex = jnp.exp(logits - m)
    s = jnp.sum(ex, axis=-1, keepdims=True)
    p = ex / s

    j = jax.lax.broadcasted_iota(jnp.int32, p.shape, 1)
    valid = j < _T
    maxp = jnp.max(p, axis=-1, keepdims=True)
    minp = jnp.min(jnp.where(valid, p, jnp.inf), axis=-1, keepdims=True)
    a = jnp.min(jnp.where(p == maxp, j, _LANES), axis=-1, keepdims=True)
    l = jnp.max(jnp.where(valid & (p == minp), j, -1), axis=-1, keepdims=True)

    use_topk = maxp < thr_ref[0]
    w_topk = jnp.where(j == l, 0.0, p)
    w_arg = jnp.where(j == a, 1.0, 0.0)
    w = jnp.where(use_topk, w_topk, w_arg).astype(jnp.bfloat16)

    e_ref[...] = jax.lax.dot_general(w, cb_ref[...], (((1,), (0,)), ((), ())),
                                     preferred_element_type=jnp.float32)
    g_ref[...] = p[:, :_T]


@jax.jit
def kernel(h_t, W1, b1, W2, b2, codebook, threshold=0.7):
    B, d_model = h_t.shape
    hidden = W1.shape[1]
    grid = B // _BB

    w2p = jnp.zeros((hidden, _LANES), jnp.float32).at[:, :_T].set(W2)
    b2p = jnp.full((1, _LANES), -1e30, jnp.float32).at[0, :_T].set(b2)
    cbp = (jnp.zeros((_LANES, d_model), jnp.float32).at[:_T, :].set(codebook)
           .astype(jnp.bfloat16))
    thr = jnp.reshape(jnp.asarray(threshold, jnp.float32), (1,))

    e_task, g_task = pl.pallas_call(
        _router_block,
        grid=(grid,),
        in_specs=[
            pl.BlockSpec(memory_space=pltpu.SMEM),
            pl.BlockSpec((_BB, d_model), lambda i: (i, 0)),
            pl.BlockSpec((d_model, hidden), lambda i: (0, 0)),
            pl.BlockSpec((1, hidden), lambda i: (0, 0)),
            pl.BlockSpec((hidden, _LANES), lambda i: (0, 0)),
            pl.BlockSpec((1, _LANES), lambda i: (0, 0)),
            pl.BlockSpec((_LANES, d_model), lambda i: (0, 0)),
        ],
        out_specs=[
            pl.BlockSpec((_BB, d_model), lambda i: (i, 0)),
            pl.BlockSpec((_BB, _T), lambda i: (i, 0)),
        ],
        out_shape=[
            jax.ShapeDtypeStruct((B, d_model), jnp.float32),
            jax.ShapeDtypeStruct((B, _T), jnp.float32),
        ],
        compiler_params=pltpu.CompilerParams(
            dimension_semantics=("parallel",),
        ),
    )(thr, h_t, W1, jnp.reshape(b1, (1, hidden)), w2p, b2p, cbp)
    return (e_task, g_task)
